# Initial kernel scaffold; baseline (speedup 1.0000x reference)
import jax
import jax.numpy as jnp
from jax.experimental import pallas as pl


def kernel(edge_index, input_features, W0, b0, g0, be0, W1, b1, g1, be1, W2, b2):
    raise NotImplementedError("write your pallas kernel here")


# speedup vs baseline: 4.6108x; 4.6108x over previous
import functools

import jax
import jax.numpy as jnp
from jax import lax
from jax.experimental import pallas as pl
from jax.experimental.pallas import tpu as pltpu
from jax.experimental.pallas import tpu_sc as plsc

EPS = 1e-5

NC = 2
NS = 16
CHUNK = 80


def _mesh():
    return plsc.VectorSubcoreMesh(core_axis_name="c", subcore_axis_name="s",
                                  num_cores=NC, num_subcores=NS)


def _deg_kernel(n_pad, e, src_hbm, dst_hbm, od_out, id_out,
                sidx, didx, ones_v, zbuf, acc_od, acc_id):
    c = lax.axis_index("c")
    s = lax.axis_index("s")
    e_pw = e // (NC * NS)
    rows_pt = n_pad // NS
    wid = s * NC + c
    base_e = wid * e_pw
    nchunk = e_pw // CHUNK

    @pl.loop(0, rows_pt // 16)
    def _z(i):
        zbuf[pl.ds(i * 16, 16)] = jnp.zeros((16,), jnp.float32)

    @pl.loop(0, CHUNK // 16)
    def _o(i):
        ones_v[pl.ds(i * 16, 16)] = jnp.ones((16,), jnp.float32)

    pltpu.sync_copy(zbuf, acc_od.at[pl.ds(s * rows_pt, rows_pt)])
    pltpu.sync_copy(zbuf, acc_id.at[pl.ds(s * rows_pt, rows_pt)])
    plsc.subcore_barrier()

    @pl.loop(0, nchunk)
    def _chunk(k):
        base = base_e + k * CHUNK
        pltpu.sync_copy(src_hbm.at[pl.ds(base, CHUNK)], sidx)
        pltpu.sync_copy(dst_hbm.at[pl.ds(base, CHUNK)], didx)
        pltpu.sync_copy(ones_v, acc_od.at[sidx], add=True)
        pltpu.sync_copy(ones_v, acc_id.at[didx], add=True)

    plsc.subcore_barrier()
    pltpu.sync_copy(acc_od.at[pl.ds(s * rows_pt, rows_pt)],
                    od_out.at[c, pl.ds(s * rows_pt, rows_pt)])
    pltpu.sync_copy(acc_id.at[pl.ds(s * rows_pt, rows_pt)],
                    id_out.at[c, pl.ds(s * rows_pt, rows_pt)])


def _make_deg(n_pad, e):
    return pl.kernel(
        functools.partial(_deg_kernel, n_pad, e),
        out_type=(jax.ShapeDtypeStruct((NC, n_pad), jnp.float32),
                  jax.ShapeDtypeStruct((NC, n_pad), jnp.float32)),
        mesh=_mesh(),
        compiler_params=pltpu.CompilerParams(use_tc_tiling_on_sc=False),
        scratch_types=[
            pltpu.VMEM((CHUNK,), jnp.int32),
            pltpu.VMEM((CHUNK,), jnp.int32),
            pltpu.VMEM((CHUNK,), jnp.float32),
            pltpu.VMEM((n_pad // NS,), jnp.float32),
            pltpu.VMEM_SHARED((n_pad,), jnp.float32),
            pltpu.VMEM_SHARED((n_pad,), jnp.float32),
        ],
    )


def _agg_kernel(n_pad, e, h_dim, h_hbm, src_hbm, dst_hbm, out,
                sidx, didx, rows, zbuf, acc, sem):
    c = lax.axis_index("c")
    s = lax.axis_index("s")
    e_pw = e // (NC * NS)
    rows_pt = n_pad // NS
    zrows = zbuf.shape[0]
    wid = s * NC + c
    base_e = wid * e_pw
    nchunk = e_pw // CHUNK

    @pl.loop(0, zrows)
    def _zr(r):
        @pl.loop(0, h_dim // 16)
        def _zc(i):
            zbuf[r, pl.ds(i * 16, 16)] = jnp.zeros((16,), jnp.float32)

    @pl.loop(0, rows_pt // zrows)
    def _zcopy(j):
        pltpu.sync_copy(zbuf, acc.at[pl.ds(s * rows_pt + j * zrows, zrows)])

    plsc.subcore_barrier()

    @pl.loop(0, nchunk)
    def _chunk(k):
        base = base_e + k * CHUNK
        pltpu.sync_copy(src_hbm.at[pl.ds(base, CHUNK)], sidx)
        pltpu.sync_copy(dst_hbm.at[pl.ds(base, CHUNK)], didx)
        pltpu.async_copy(h_hbm.at[sidx], rows, sem).wait()
        pltpu.sync_copy(rows, acc.at[didx], add=True)

    plsc.subcore_barrier()
    pltpu.sync_copy(acc.at[pl.ds(s * rows_pt, rows_pt)],
                    out.at[c, pl.ds(s * rows_pt, rows_pt)])


def _make_agg(n_pad, e, h_dim):
    zrows = 128
    return pl.kernel(
        functools.partial(_agg_kernel, n_pad, e, h_dim),
        out_type=jax.ShapeDtypeStruct((NC, n_pad, h_dim), jnp.float32),
        mesh=_mesh(),
        compiler_params=pltpu.CompilerParams(use_tc_tiling_on_sc=False),
        scratch_types=[
            pltpu.VMEM((CHUNK,), jnp.int32),
            pltpu.VMEM((CHUNK,), jnp.int32),
            pltpu.VMEM((CHUNK, h_dim), jnp.float32),
            pltpu.VMEM((zrows, h_dim), jnp.float32),
            pltpu.VMEM_SHARED((n_pad, h_dim), jnp.float32),
            pltpu.SemaphoreType.DMA,
        ],
    )


def _tc_call(body, out_shape, n_in):
    return pl.pallas_call(
        body,
        out_shape=out_shape,
        in_specs=[pl.BlockSpec(memory_space=pltpu.VMEM)] * n_in,
        out_specs=pl.BlockSpec(memory_space=pltpu.VMEM),
    )


def _first_kernel(n, x_ref, w_ref, dod_ref, out_ref):
    dout = dod_ref[0, :n] + dod_ref[1, :n]
    r = lax.rsqrt(jnp.maximum(dout, 1.0))
    out_ref[...] = jnp.dot(x_ref[...] * r[:, None], w_ref[...],
                           preferred_element_type=jnp.float32)


def _mid_kernel(n, p_ref, did_ref, dod_ref, b_ref, g_ref, be_ref, w_ref,
                out_ref):
    p = p_ref[0, :n, :] + p_ref[1, :n, :]
    din = did_ref[0, :n] + did_ref[1, :n]
    y = p * lax.rsqrt(jnp.maximum(din, 1.0))[:, None] + b_ref[...]
    mean = jnp.mean(y, axis=0, keepdims=True)
    var = jnp.mean((y - mean) ** 2, axis=0, keepdims=True)
    z = g_ref[...] * (y - mean) / jnp.sqrt(var + EPS) + be_ref[...]
    z = jnp.maximum(z, 0.0)
    dout = dod_ref[0, :n] + dod_ref[1, :n]
    z = z * lax.rsqrt(jnp.maximum(dout, 1.0))[:, None]
    out_ref[...] = jnp.dot(z, w_ref[...], preferred_element_type=jnp.float32)


def _last_kernel(n, p_ref, did_ref, b_ref, out_ref):
    p = p_ref[0, :n, :] + p_ref[1, :n, :]
    din = did_ref[0, :n] + did_ref[1, :n]
    out_ref[...] = (p * lax.rsqrt(jnp.maximum(din, 1.0))[:, None]
                    + b_ref[...])


def kernel(edge_index, input_features, W0, b0, g0, be0, W1, b1, g1, be1,
           W2, b2):
    n, d_in = input_features.shape
    e = edge_index.shape[1]
    h = W0.shape[1]
    d_out = W2.shape[1]
    n_pad = ((n + 8 * NS - 1) // (8 * NS)) * (8 * NS)

    src = edge_index[0]
    dst = edge_index[1]

    deg = _make_deg(n_pad, e)
    agg_h = _make_agg(n_pad, e, h)
    agg_o = _make_agg(n_pad, e, d_out)

    od_p, id_p = deg(src, dst)

    b0r, g0r, be0r = b0.reshape(1, -1), g0.reshape(1, -1), be0.reshape(1, -1)
    b1r, g1r, be1r = b1.reshape(1, -1), g1.reshape(1, -1), be1.reshape(1, -1)
    b2r = b2.reshape(1, -1)

    h0 = _tc_call(functools.partial(_first_kernel, n),
                  jax.ShapeDtypeStruct((n, h), jnp.float32), 3)(
                      input_features, W0, od_p)
    p0 = agg_h(h0, src, dst)
    h1 = _tc_call(functools.partial(_mid_kernel, n),
                  jax.ShapeDtypeStruct((n, h), jnp.float32), 7)(
                      p0, id_p, od_p, b0r, g0r, be0r, W1)
    p1 = agg_h(h1, src, dst)
    h2 = _tc_call(functools.partial(_mid_kernel, n),
                  jax.ShapeDtypeStruct((n, d_out), jnp.float32), 7)(
                      p1, id_p, od_p, b1r, g1r, be1r, W2)
    p2 = agg_o(h2, src, dst)
    out = _tc_call(functools.partial(_last_kernel, n),
                   jax.ShapeDtypeStruct((n, d_out), jnp.float32), 3)(
                       p2, id_p, b2r)
    return out


# speedup vs baseline: 11.6936x; 2.5361x over previous
import functools

import jax
import jax.numpy as jnp
from jax import lax
from jax.experimental import pallas as pl
from jax.experimental.pallas import tpu as pltpu
from jax.experimental.pallas import tpu_sc as plsc

EPS = 1e-5

NC = 2
NS = 16
CHUNK = 80


def _mesh():
    return plsc.VectorSubcoreMesh(core_axis_name="c", subcore_axis_name="s",
                                  num_cores=NC, num_subcores=NS)


def _sc_params():
    return pltpu.CompilerParams(use_tc_tiling_on_sc=False)


def _deg_kernel(n_pad, e, src2_hbm, dst2_hbm, od_out, id_out,
                sidx, didx, ones_v, zbuf, acc_od, acc_id, sem):
    c = lax.axis_index("c")
    s = lax.axis_index("s")
    rows_pt = n_pad // NS
    nchunk = (e // (NC * NS)) // CHUNK
    wid = s * NC + c
    base_k = wid * nchunk

    @pl.loop(0, rows_pt // 16)
    def _z(i):
        zbuf[pl.ds(i * 16, 16)] = jnp.zeros((16,), jnp.float32)

    @pl.loop(0, CHUNK // 16)
    def _o(i):
        ones_v[pl.ds(i * 16, 16)] = jnp.ones((16,), jnp.float32)

    pltpu.sync_copy(src2_hbm.at[pl.ds(base_k, nchunk)], sidx)
    pltpu.sync_copy(dst2_hbm.at[pl.ds(base_k, nchunk)], didx)
    pltpu.sync_copy(zbuf, acc_od.at[pl.ds(s * rows_pt, rows_pt)])
    pltpu.sync_copy(zbuf, acc_id.at[pl.ds(s * rows_pt, rows_pt)])
    plsc.subcore_barrier()

    def _scat(k):
        pltpu.async_copy(ones_v, acc_od.at[sidx.at[k]], sem, add=True)
        pltpu.async_copy(ones_v, acc_id.at[didx.at[k]], sem, add=True)

    def _drain(k):
        pltpu.make_async_copy(ones_v, acc_od.at[sidx.at[k]], sem).wait()
        pltpu.make_async_copy(ones_v, acc_id.at[didx.at[k]], sem).wait()

    _scat(0)

    @pl.loop(1, nchunk)
    def _chunk(k):
        _scat(k)
        _drain(k)

    _drain(0)
    plsc.subcore_barrier()
    pltpu.sync_copy(acc_od.at[pl.ds(s * rows_pt, rows_pt)],
                    od_out.at[c, pl.ds(s * rows_pt, rows_pt)])
    pltpu.sync_copy(acc_id.at[pl.ds(s * rows_pt, rows_pt)],
                    id_out.at[c, pl.ds(s * rows_pt, rows_pt)])


def _make_deg(n_pad, e):
    nchunk = (e // (NC * NS)) // CHUNK
    return pl.kernel(
        functools.partial(_deg_kernel, n_pad, e),
        out_type=(jax.ShapeDtypeStruct((NC, n_pad), jnp.float32),
                  jax.ShapeDtypeStruct((NC, n_pad), jnp.float32)),
        mesh=_mesh(),
        compiler_params=_sc_params(),
        scratch_types=[
            pltpu.VMEM((nchunk, CHUNK), jnp.int32),
            pltpu.VMEM((nchunk, CHUNK), jnp.int32),
            pltpu.VMEM((CHUNK,), jnp.float32),
            pltpu.VMEM((n_pad // NS,), jnp.float32),
            pltpu.VMEM_SHARED((n_pad,), jnp.float32),
            pltpu.VMEM_SHARED((n_pad,), jnp.float32),
            pltpu.SemaphoreType.DMA,
        ],
    )


def _agg_kernel(n_pad, e, h_dim, h_hbm, src2_hbm, dst2_hbm, out,
                sidx, didx, rows, zbuf, acc, sem0, sem1):
    c = lax.axis_index("c")
    s = lax.axis_index("s")
    rows_pt = n_pad // NS
    zrows = zbuf.shape[0]
    nchunk = (e // (NC * NS)) // CHUNK
    wid = s * NC + c
    base_k = wid * nchunk

    @pl.loop(0, zrows)
    def _zr(r):
        @pl.loop(0, h_dim // 16)
        def _zc(i):
            zbuf[r, pl.ds(i * 16, 16)] = jnp.zeros((16,), jnp.float32)

    pltpu.sync_copy(src2_hbm.at[pl.ds(base_k, nchunk)], sidx)
    pltpu.sync_copy(dst2_hbm.at[pl.ds(base_k, nchunk)], didx)

    @pl.loop(0, rows_pt // zrows)
    def _zcopy(j):
        pltpu.sync_copy(zbuf, acc.at[pl.ds(s * rows_pt + j * zrows, zrows)])

    plsc.subcore_barrier()

    sems = (sem0, sem1)

    def _gather(k, b):
        pltpu.async_copy(h_hbm.at[sidx.at[k]], rows.at[b], sems[b])

    def _gwait(k, b):
        pltpu.make_async_copy(h_hbm.at[sidx.at[k]], rows.at[b],
                              sems[b]).wait()

    def _scat(k, b):
        pltpu.sync_copy(rows.at[b], acc.at[didx.at[k]], add=True)

    _gather(0, 0)

    @pl.loop(0, (nchunk - 1) // 2)
    def _pipe(g):
        k = 2 * g
        _gather(k + 1, 1)
        _gwait(k, 0)
        _scat(k, 0)
        _gather(k + 2, 0)
        _gwait(k + 1, 1)
        _scat(k + 1, 1)

    _gwait(nchunk - 1, 0)
    _scat(nchunk - 1, 0)

    plsc.subcore_barrier()
    pltpu.sync_copy(acc.at[pl.ds(s * rows_pt, rows_pt)],
                    out.at[c, pl.ds(s * rows_pt, rows_pt)])


def _make_agg(n_pad, e, h_dim):
    zrows = 8
    nchunk = (e // (NC * NS)) // CHUNK
    return pl.kernel(
        functools.partial(_agg_kernel, n_pad, e, h_dim),
        out_type=jax.ShapeDtypeStruct((NC, n_pad, h_dim), jnp.float32),
        mesh=_mesh(),
        compiler_params=_sc_params(),
        scratch_types=[
            pltpu.VMEM((nchunk, CHUNK), jnp.int32),
            pltpu.VMEM((nchunk, CHUNK), jnp.int32),
            pltpu.VMEM((2, CHUNK, h_dim), jnp.float32),
            pltpu.VMEM((zrows, h_dim), jnp.float32),
            pltpu.VMEM_SHARED((n_pad, h_dim), jnp.float32),
            pltpu.SemaphoreType.DMA,
            pltpu.SemaphoreType.DMA,
        ],
    )


def _tc_call(body, out_shape, n_in):
    return pl.pallas_call(
        body,
        out_shape=out_shape,
        in_specs=[pl.BlockSpec(memory_space=pltpu.VMEM)] * n_in,
        out_specs=pl.BlockSpec(memory_space=pltpu.VMEM),
    )


def _first_kernel(n, x_ref, w_ref, dod_ref, out_ref):
    dout = dod_ref[0, :n] + dod_ref[1, :n]
    r = lax.rsqrt(jnp.maximum(dout, 1.0))
    out_ref[...] = jnp.dot(x_ref[...] * r[:, None], w_ref[...],
                           preferred_element_type=jnp.float32)


def _mid_kernel(n, p_ref, did_ref, dod_ref, b_ref, g_ref, be_ref, w_ref,
                out_ref):
    p = p_ref[0, :n, :] + p_ref[1, :n, :]
    din = did_ref[0, :n] + did_ref[1, :n]
    y = p * lax.rsqrt(jnp.maximum(din, 1.0))[:, None] + b_ref[...]
    mean = jnp.mean(y, axis=0, keepdims=True)
    var = jnp.mean((y - mean) ** 2, axis=0, keepdims=True)
    z = g_ref[...] * (y - mean) / jnp.sqrt(var + EPS) + be_ref[...]
    z = jnp.maximum(z, 0.0)
    dout = dod_ref[0, :n] + dod_ref[1, :n]
    z = z * lax.rsqrt(jnp.maximum(dout, 1.0))[:, None]
    out_ref[...] = jnp.dot(z, w_ref[...], preferred_element_type=jnp.float32)


def _last_kernel(n, p_ref, did_ref, b_ref, out_ref):
    p = p_ref[0, :n, :] + p_ref[1, :n, :]
    din = did_ref[0, :n] + did_ref[1, :n]
    out_ref[...] = (p * lax.rsqrt(jnp.maximum(din, 1.0))[:, None]
                    + b_ref[...])


def kernel(edge_index, input_features, W0, b0, g0, be0, W1, b1, g1, be1,
           W2, b2):
    n, d_in = input_features.shape
    e = edge_index.shape[1]
    h = W0.shape[1]
    d_out = W2.shape[1]
    n_pad = ((n + 8 * NS - 1) // (8 * NS)) * (8 * NS)

    src2 = edge_index[0].reshape(e // CHUNK, CHUNK)
    dst2 = edge_index[1].reshape(e // CHUNK, CHUNK)

    deg = _make_deg(n_pad, e)
    agg_h = _make_agg(n_pad, e, h)
    agg_o = _make_agg(n_pad, e, d_out)

    od_p, id_p = deg(src2, dst2)

    b0r, g0r, be0r = b0.reshape(1, -1), g0.reshape(1, -1), be0.reshape(1, -1)
    b1r, g1r, be1r = b1.reshape(1, -1), g1.reshape(1, -1), be1.reshape(1, -1)
    b2r = b2.reshape(1, -1)

    h0 = _tc_call(functools.partial(_first_kernel, n),
                  jax.ShapeDtypeStruct((n, h), jnp.float32), 3)(
                      input_features, W0, od_p)
    p0 = agg_h(h0, src2, dst2)
    h1 = _tc_call(functools.partial(_mid_kernel, n),
                  jax.ShapeDtypeStruct((n, h), jnp.float32), 7)(
                      p0, id_p, od_p, b0r, g0r, be0r, W1)
    p1 = agg_h(h1, src2, dst2)
    h2 = _tc_call(functools.partial(_mid_kernel, n),
                  jax.ShapeDtypeStruct((n, d_out), jnp.float32), 7)(
                      p1, id_p, od_p, b1r, g1r, be1r, W2)
    p2 = agg_o(h2, src2, dst2)
    out = _tc_call(functools.partial(_last_kernel, n),
                   jax.ShapeDtypeStruct((n, d_out), jnp.float32), 3)(
                       p2, id_p, b2r)
    return out
